# baseline (speedup 1.0000x reference)
import functools

import jax
import jax.numpy as jnp
from jax import lax
from jax.experimental import pallas as pl
from jax.experimental.pallas import tpu as pltpu
from jax.experimental.pallas import tpu_sc as plsc

_B = 16384
_D = 64
_NC = 2
_NS = 16
_NW = _NC * _NS
_LANES = 16
_TCOLS = 7813
_COLS = 245
_NB = (_COLS + 1) // 2
_LMAX = 640
_CL = 2048
_WR = 16


def _sc_gather(labels1, cen_t):
    mesh = plsc.VectorSubcoreMesh(core_axis_name="c", subcore_axis_name="s")

    @functools.partial(
        pl.kernel,
        mesh=mesh,
        out_type=jax.ShapeDtypeStruct((_B, _D), jnp.float32),
        scratch_types=[
            pltpu.VMEM((2, _CL), jnp.int32),
            pltpu.VMEM((_LMAX + _LANES,), jnp.int32),
            pltpu.VMEM((_NB + 2 * _LANES,), jnp.int32),
            pltpu.VMEM((5, _D, 256), jnp.float32),
            pltpu.VMEM((_WR, 1, _D), jnp.float32),
            pltpu.SMEM((_NB + 1,), jnp.int32),
            pltpu.SMEM((_NB,), jnp.int32),
            pltpu.SMEM((_LMAX,), jnp.int32),
        ] + [pltpu.SemaphoreType.DMA] * 7,
        compiler_params=pltpu.CompilerParams(use_tc_tiling_on_sc=True,
                                             needs_layout_passes=False),
    )
    def k(labels_hbm, cen_hbm, out_hbm, lab_v, sel_v, cnt_v, blk_v, row_v,
          off_s, cur_s, bin_s, bsem0, bsem1, bsem2, bsem3, bsem4, lsem, wsem):
        cid = lax.axis_index("c")
        sid = lax.axis_index("s")
        wid = sid * _NC + cid
        col0 = wid * _COLS
        bsems = [bsem0, bsem1, bsem2, bsem3, bsem4]

        def fire(q, b):
            off = jnp.minimum((col0 + 2 * q) * 128, (_TCOLS - 2) * 128)
            off = pl.multiple_of(off, 128)
            for kk in range(_D // 8):
                pltpu.async_copy(
                    cen_hbm.at[pl.ds(8 * kk, 8), pl.ds(off, 256)],
                    blk_v.at[b, pl.ds(8 * kk, 8)], bsems[b])

        def drain(b):
            for kk in range(_D // 8):
                pltpu.make_async_copy(
                    cen_hbm.at[pl.ds(0, 8), pl.ds(0, 256)],
                    blk_v.at[b, pl.ds(8 * kk, 8)], bsems[b]).wait()

        for b in range(5):
            fire(b, b)

        zero = jnp.zeros((_LANES,), jnp.int32)
        for v in range((_NB + 2 * _LANES) // _LANES):
            cnt_v[pl.ds(v * _LANES, _LANES)] = zero

        one = jnp.full((_LANES,), 1, jnp.int32)
        lane = lax.iota(jnp.int32, _LANES)

        def lfire(c):
            pltpu.async_copy(
                labels_hbm.at[pl.ds(c * _CL, _CL)], lab_v.at[c % 2], lsem)

        lfire(0)
        lfire(1)
        nvec = zero
        for c in range(_B // _CL):
            pltpu.make_async_copy(
                labels_hbm.at[pl.ds(0, _CL)], lab_v.at[c % 2], lsem).wait()

            def filt(v, n, _c=c):
                lv = lab_v[_c % 2, pl.ds(v * _LANES, _LANES)]
                rel = (lv >> 7) - col0
                mask = (rel >= 0) & (rel < _COLS)
                bq = rel >> 1
                cc = lv - (col0 * 128) - (bq << 8)
                iv = lane + (_c * _CL + v * _LANES)
                pack = bq | (cc << 7) | (iv << 15)
                mi = mask.astype(jnp.int32)
                pos = n + plsc.cumsum(mi) - 1
                plsc.store_scatter(
                    sel_v, [jnp.where(mask, pos, _LMAX)], pack)
                plsc.addupdate_scatter(
                    cnt_v, [jnp.where(mask, bq, _NB + _LANES)], one)
                return n + plsc.all_reduce_population_count(mask)

            nvec = lax.fori_loop(0, _CL // _LANES, filt, nvec, unroll=2)
            if c + 2 < _B // _CL:
                lfire(c + 2)
        num = nvec[0]

        def pref(m, run):
            off_s[m] = run
            cur_s[m] = run
            return run + cnt_v[pl.ds(m, _LANES)][0]

        total = lax.fori_loop(0, _NB, pref, jnp.int32(0))
        off_s[_NB] = total

        def binp(p, carry):
            v = sel_v[pl.ds(p, _LANES)][0]
            bq = v & 127
            q = cur_s[bq]
            cur_s[bq] = q + 1
            bin_s[q] = v >> 7
            return carry

        lax.fori_loop(0, num, binp, jnp.int32(0))

        idx0 = [lane + t * _LANES for t in range(_D // _LANES)]

        def scan(g, h):
            for b in range(5):
                q = 5 * g + b
                drain(b)
                start = off_s[jnp.minimum(q, _NB)]
                end = off_s[jnp.minimum(q + 1, _NB)]

                def hit(carry):
                    qq, hh = carry
                    v = bin_s[qq]
                    cc = jax.lax.broadcast(v & 255, (_LANES,))
                    i = v >> 8
                    rr = hh % _WR

                    @pl.when(hh >= _WR)
                    def _():
                        pltpu.make_async_copy(
                            row_v.at[0], out_hbm.at[pl.ds(0, 1)],
                            wsem).wait()

                    for t in range(_D // _LANES):
                        row_v[rr, 0, pl.ds(t * _LANES, _LANES)] = (
                            plsc.load_gather(blk_v.at[b], [idx0[t], cc]))
                    pltpu.async_copy(
                        row_v.at[rr], out_hbm.at[pl.ds(i, 1)], wsem)
                    return qq + 1, hh + 1

                _, h = lax.while_loop(
                    lambda carry: carry[0] < end, hit, (start, h))
                fire(q + 5, b)
            return h

        h = lax.fori_loop(0, _NB // 5 + 1, scan, jnp.int32(0))
        for b in range(5):
            drain(b)
        lax.fori_loop(
            0, jnp.minimum(h, _WR),
            lambda p, carry: (pltpu.make_async_copy(
                row_v.at[0], out_hbm.at[pl.ds(0, 1)], wsem).wait(), carry)[1],
            jnp.int32(0))

    return k(labels1, cen_t)


def _finish(ft_ref, g_ref, o_ref):
    ft = ft_ref[...]
    g = g_ref[...]
    prod = jax.lax.dot(ft, g, precision=jax.lax.Precision.HIGHEST)
    r = jax.lax.broadcasted_iota(jnp.int32, (_D, _D), 0)
    c = jax.lax.broadcasted_iota(jnp.int32, (_D, _D), 1)
    tr = jnp.sum(jnp.where(r == c, prod, 0.0))
    o_ref[0] = (jnp.sum(ft * ft) + jnp.sum(g * g) - 2.0 * tr) * (0.5 / _B)


def kernel(features, labels, centers):
    labels1 = labels.astype(jnp.int32)
    gathered = _sc_gather(labels1, centers.T)
    loss = pl.pallas_call(
        _finish,
        out_shape=jax.ShapeDtypeStruct((1,), jnp.float32),
        out_specs=pl.BlockSpec(memory_space=pltpu.SMEM),
    )(features.T, gathered)
    return loss[0]


# speedup vs baseline: 1.0162x; 1.0162x over previous
import functools

import jax
import jax.numpy as jnp
from jax import lax
from jax.experimental import pallas as pl
from jax.experimental.pallas import tpu as pltpu
from jax.experimental.pallas import tpu_sc as plsc

_B = 16384
_D = 64
_NC = 2
_NS = 16
_NW = _NC * _NS
_LANES = 16
_TCOLS = 7813
_COLS = 245
_NB = (_COLS + 1) // 2
_LMAX = 640
_CL = 2048
_WR = 16


def _sc_gather(labels1, cen_t):
    mesh = plsc.VectorSubcoreMesh(core_axis_name="c", subcore_axis_name="s")

    @functools.partial(
        pl.kernel,
        mesh=mesh,
        out_type=jax.ShapeDtypeStruct((_B, _D), jnp.float32),
        scratch_types=[
            pltpu.VMEM((2, _CL), jnp.int32),
            pltpu.VMEM((_LMAX + _LANES,), jnp.int32),
            pltpu.VMEM((_NB + 2 * _LANES,), jnp.int32),
            pltpu.VMEM((5, _D, 256), jnp.float32),
            pltpu.VMEM((_WR, 1, _D), jnp.float32),
            pltpu.SMEM((_NB + 1,), jnp.int32),
            pltpu.SMEM((_NB,), jnp.int32),
            pltpu.SMEM((_LMAX,), jnp.int32),
        ] + [pltpu.SemaphoreType.DMA] * 7,
        compiler_params=pltpu.CompilerParams(use_tc_tiling_on_sc=True,
                                             needs_layout_passes=False),
    )
    def k(labels_hbm, cen_hbm, out_hbm, lab_v, sel_v, cnt_v, blk_v, row_v,
          off_s, cur_s, bin_s, bsem0, bsem1, bsem2, bsem3, bsem4, lsem, wsem):
        cid = lax.axis_index("c")
        sid = lax.axis_index("s")
        wid = sid * _NC + cid
        col0 = wid * _COLS
        bsems = [bsem0, bsem1, bsem2, bsem3, bsem4]

        def fire(q, b):
            off = jnp.minimum((col0 + 2 * q) * 128, (_TCOLS - 2) * 128)
            off = pl.multiple_of(off, 128)
            for kk in range(_D // 8):
                pltpu.async_copy(
                    cen_hbm.at[pl.ds(8 * kk, 8), pl.ds(off, 256)],
                    blk_v.at[b, pl.ds(8 * kk, 8)], bsems[b])

        def drain(b):
            for kk in range(_D // 8):
                pltpu.make_async_copy(
                    cen_hbm.at[pl.ds(0, 8), pl.ds(0, 256)],
                    blk_v.at[b, pl.ds(8 * kk, 8)], bsems[b]).wait()

        for b in range(5):
            fire(b, b)

        zero = jnp.zeros((_LANES,), jnp.int32)
        for v in range((_NB + 2 * _LANES) // _LANES):
            cnt_v[pl.ds(v * _LANES, _LANES)] = zero

        one = jnp.full((_LANES,), 1, jnp.int32)
        lane = lax.iota(jnp.int32, _LANES)

        def lfire(c):
            pltpu.async_copy(
                labels_hbm.at[pl.ds(c * _CL, _CL)], lab_v.at[c % 2], lsem)

        lfire(0)
        lfire(1)
        nvec = zero
        for c in range(_B // _CL):
            pltpu.make_async_copy(
                labels_hbm.at[pl.ds(0, _CL)], lab_v.at[c % 2], lsem).wait()

            def filt(v, n, _c=c):
                lv = lab_v[_c % 2, pl.ds(v * _LANES, _LANES)]
                rel = (lv >> 7) - col0
                mask = (rel >= 0) & (rel < _COLS)
                bq = rel >> 1
                cc = lv - (col0 * 128) - (bq << 8)
                iv = lane + (_c * _CL + v * _LANES)
                pack = bq | (cc << 7) | (iv << 15)
                mi = mask.astype(jnp.int32)
                pos = n + plsc.cumsum(mi) - 1
                plsc.store_scatter(
                    sel_v, [jnp.where(mask, pos, _LMAX)], pack)
                plsc.addupdate_scatter(
                    cnt_v, [jnp.where(mask, bq, _NB + _LANES)], one)
                return n + plsc.all_reduce_population_count(mask)

            nvec = lax.fori_loop(0, _CL // _LANES, filt, nvec, unroll=2)
            if c + 2 < _B // _CL:
                lfire(c + 2)
        num = nvec[0]

        def pref(m, run):
            off_s[m] = run
            cur_s[m] = run
            return run + cnt_v[pl.ds(m, _LANES)][0]

        total = lax.fori_loop(0, _NB, pref, jnp.int32(0))
        off_s[_NB] = total

        def binp(p, carry):
            v = sel_v[pl.ds(p, _LANES)][0]
            bq = v & 127
            q = cur_s[bq]
            cur_s[bq] = q + 1
            bin_s[q] = v >> 7
            return carry

        lax.fori_loop(0, num, binp, jnp.int32(0))

        idx0 = [lane + t * _LANES for t in range(_D // _LANES)]

        def scan(g, h):
            for b in range(5):
                q = 5 * g + b
                drain(b)
                start = off_s[jnp.minimum(q, _NB)]
                end = off_s[jnp.minimum(q + 1, _NB)]

                def hit(carry):
                    qq, hh = carry
                    v = bin_s[qq]
                    cc = jax.lax.broadcast(v & 255, (_LANES,))
                    i = v >> 8
                    rr = hh % _WR

                    @pl.when(hh >= _WR)
                    def _():
                        pltpu.make_async_copy(
                            row_v.at[0], out_hbm.at[pl.ds(0, 1)],
                            wsem).wait()

                    for t in range(_D // _LANES):
                        row_v[rr, 0, pl.ds(t * _LANES, _LANES)] = (
                            plsc.load_gather(blk_v.at[b], [idx0[t], cc]))
                    pltpu.async_copy(
                        row_v.at[rr], out_hbm.at[pl.ds(i, 1)], wsem)
                    return qq + 1, hh + 1

                _, h = lax.while_loop(
                    lambda carry: carry[0] < end, hit, (start, h))
                fire(q + 5, b)
            return h

        h = lax.fori_loop(0, _NB // 5 + 1, scan, jnp.int32(0))
        for b in range(5):
            drain(b)
        lax.fori_loop(
            0, jnp.minimum(h, _WR),
            lambda p, carry: (pltpu.make_async_copy(
                row_v.at[0], out_hbm.at[pl.ds(0, 1)], wsem).wait(), carry)[1],
            jnp.int32(0))

    return k(labels1, cen_t)


def _finish(f_ref, g_ref, o_ref):
    d = f_ref[...] - g_ref[...]
    o_ref[0] = jnp.sum(d * d) * (0.5 / _B)


def kernel(features, labels, centers):
    labels1 = labels.astype(jnp.int32)
    gathered = _sc_gather(labels1, centers.T)
    loss = pl.pallas_call(
        _finish,
        out_shape=jax.ShapeDtypeStruct((1,), jnp.float32),
        out_specs=pl.BlockSpec(memory_space=pltpu.SMEM),
    )(features, gathered)
    return loss[0]


# speedup vs baseline: 1.0302x; 1.0138x over previous
import functools

import jax
import jax.numpy as jnp
from jax import lax
from jax.experimental import pallas as pl
from jax.experimental.pallas import tpu as pltpu
from jax.experimental.pallas import tpu_sc as plsc

_B = 16384
_D = 64
_NC = 2
_NS = 16
_NW = _NC * _NS
_LANES = 16
_TCOLS = 7813
_COLS = 245
_NB = (_COLS + 1) // 2
_LMAX = 640
_CL = 2048
_WR = 16


def _sc_gather(labels1, cen_t):
    mesh = plsc.VectorSubcoreMesh(core_axis_name="c", subcore_axis_name="s")

    @functools.partial(
        pl.kernel,
        mesh=mesh,
        out_type=jax.ShapeDtypeStruct((_B, _D), jnp.float32),
        scratch_types=[
            pltpu.VMEM((2, _CL), jnp.int32),
            pltpu.VMEM((_LMAX + _LANES,), jnp.int32),
            pltpu.VMEM((_NB + 2 * _LANES,), jnp.int32),
            pltpu.VMEM((5, _D, 256), jnp.float32),
            pltpu.VMEM((_WR, 1, _D), jnp.float32),
            pltpu.VMEM((_NB + 2 * _LANES,), jnp.int32),
            pltpu.VMEM((_LMAX + _LANES,), jnp.int32),
            pltpu.SMEM((_NB + 1,), jnp.int32),
        ] + [pltpu.SemaphoreType.DMA] * 7,
        compiler_params=pltpu.CompilerParams(use_tc_tiling_on_sc=True,
                                             needs_layout_passes=False),
    )
    def k(labels_hbm, cen_hbm, out_hbm, lab_v, sel_v, cnt_v, blk_v, row_v,
          cur_v, bin_v, off_s, bsem0, bsem1, bsem2, bsem3, bsem4, lsem, wsem):
        cid = lax.axis_index("c")
        sid = lax.axis_index("s")
        wid = sid * _NC + cid
        col0 = wid * _COLS
        bsems = [bsem0, bsem1, bsem2, bsem3, bsem4]

        def fire(q, b):
            off = jnp.minimum((col0 + 2 * q) * 128, (_TCOLS - 2) * 128)
            off = pl.multiple_of(off, 128)
            for kk in range(_D // 8):
                pltpu.async_copy(
                    cen_hbm.at[pl.ds(8 * kk, 8), pl.ds(off, 256)],
                    blk_v.at[b, pl.ds(8 * kk, 8)], bsems[b])

        def drain(b):
            for kk in range(_D // 8):
                pltpu.make_async_copy(
                    cen_hbm.at[pl.ds(0, 8), pl.ds(0, 256)],
                    blk_v.at[b, pl.ds(8 * kk, 8)], bsems[b]).wait()

        for b in range(5):
            fire(b, b)

        zero = jnp.zeros((_LANES,), jnp.int32)
        for v in range((_NB + 2 * _LANES) // _LANES):
            cnt_v[pl.ds(v * _LANES, _LANES)] = zero

        one = jnp.full((_LANES,), 1, jnp.int32)
        lane = lax.iota(jnp.int32, _LANES)

        def lfire(c):
            pltpu.async_copy(
                labels_hbm.at[pl.ds(c * _CL, _CL)], lab_v.at[c % 2], lsem)

        lfire(0)
        lfire(1)
        nvec = zero
        for c in range(_B // _CL):
            pltpu.make_async_copy(
                labels_hbm.at[pl.ds(0, _CL)], lab_v.at[c % 2], lsem).wait()

            def filt(v, n, _c=c):
                lv = lab_v[_c % 2, pl.ds(v * _LANES, _LANES)]
                rel = (lv >> 7) - col0
                mask = (rel >= 0) & (rel < _COLS)
                bq = rel >> 1
                cc = lv - (col0 * 128) - (bq << 8)
                iv = lane + (_c * _CL + v * _LANES)
                pack = bq | (cc << 7) | (iv << 15)
                mi = mask.astype(jnp.int32)
                pos = n + plsc.cumsum(mi) - 1
                plsc.store_scatter(
                    sel_v, [jnp.where(mask, pos, _LMAX)], pack)
                plsc.addupdate_scatter(
                    cnt_v, [jnp.where(mask, bq, _NB + _LANES)], one)
                return n + plsc.all_reduce_population_count(mask)

            nvec = lax.fori_loop(0, _CL // _LANES, filt, nvec, unroll=2)
            if c + 2 < _B // _CL:
                lfire(c + 2)
        num = nvec[0]

        def pref(m, run):
            off_s[m] = run
            return run + cnt_v[pl.ds(m, _LANES)][0]

        total = lax.fori_loop(0, _NB, pref, jnp.int32(0))
        off_s[_NB] = total

        carry = zero
        for vg in range((_NB + 2 * _LANES) // _LANES):
            cg = cnt_v[pl.ds(vg * _LANES, _LANES)]
            inc = plsc.cumsum(cg)
            cur_v[pl.ds(vg * _LANES, _LANES)] = carry + inc - cg
            carry = carry + jax.lax.broadcast(jnp.sum(cg), (_LANES,))

        def binv(gi, c2):
            base = gi * _LANES
            vv = sel_v[pl.ds(base, _LANES)]
            valid = (lane + base) < num
            bq = jnp.where(valid, vv & 127, _NB + _LANES)
            ordv = plsc.scan_count(bq)[0] - 1
            pos = plsc.load_gather(cur_v, [bq]) + ordv
            plsc.store_scatter(
                bin_v, [jnp.where(valid, pos, _LMAX)], vv >> 7)
            plsc.addupdate_scatter(cur_v, [bq], one)
            return c2

        lax.fori_loop(0, (num + _LANES - 1) // _LANES, binv, jnp.int32(0))

        idx0 = [lane + t * _LANES for t in range(_D // _LANES)]

        def scan(g, h):
            for b in range(5):
                q = 5 * g + b
                drain(b)
                start = off_s[jnp.minimum(q, _NB)]
                end = off_s[jnp.minimum(q + 1, _NB)]

                def hit(carry):
                    qq, hh = carry
                    v = bin_v[pl.ds(qq, _LANES)][0]
                    cc = jax.lax.broadcast(v & 255, (_LANES,))
                    i = v >> 8
                    rr = hh % _WR

                    @pl.when(hh >= _WR)
                    def _():
                        pltpu.make_async_copy(
                            row_v.at[0], out_hbm.at[pl.ds(0, 1)],
                            wsem).wait()

                    for t in range(_D // _LANES):
                        row_v[rr, 0, pl.ds(t * _LANES, _LANES)] = (
                            plsc.load_gather(blk_v.at[b], [idx0[t], cc]))
                    pltpu.async_copy(
                        row_v.at[rr], out_hbm.at[pl.ds(i, 1)], wsem)
                    return qq + 1, hh + 1

                _, h = lax.while_loop(
                    lambda carry: carry[0] < end, hit, (start, h))
                fire(q + 5, b)
            return h

        h = lax.fori_loop(0, _NB // 5 + 1, scan, jnp.int32(0))
        for b in range(5):
            drain(b)
        lax.fori_loop(
            0, jnp.minimum(h, _WR),
            lambda p, carry: (pltpu.make_async_copy(
                row_v.at[0], out_hbm.at[pl.ds(0, 1)], wsem).wait(), carry)[1],
            jnp.int32(0))

    return k(labels1, cen_t)


def _finish(f_ref, g_ref, o_ref):
    d = f_ref[...] - g_ref[...]
    o_ref[0] = jnp.sum(d * d) * (0.5 / _B)


def kernel(features, labels, centers):
    labels1 = labels.astype(jnp.int32)
    gathered = _sc_gather(labels1, centers.T)
    loss = pl.pallas_call(
        _finish,
        out_shape=jax.ShapeDtypeStruct((1,), jnp.float32),
        out_specs=pl.BlockSpec(memory_space=pltpu.SMEM),
    )(features, gathered)
    return loss[0]
